# baseline (speedup 1.0000x reference)
import functools

import jax
import jax.numpy as jnp
from jax import lax
from jax.experimental import pallas as pl
from jax.experimental.pallas import tpu as pltpu
from jax.experimental.pallas import tpu_sc as plsc

_V = 1000000
_D = 64
_A = 4096
_S = 200
_SCALE = float(_D) ** 0.5
_K1_COLS = 4096


def _k1_body(t_ref, o_ref):
    blk = t_ref[...]
    t = jnp.transpose(blk, (1, 0)) * _SCALE
    t3 = jnp.reshape(t, (_K1_COLS // 2, 2, _D))
    o_ref[...] = jnp.concatenate([t3[:, 0, :], t3[:, 1, :]], axis=1)


def _k1(tT):
    n_blk = (_V + _K1_COLS - 1) // _K1_COLS
    return pl.pallas_call(
        _k1_body,
        grid=(n_blk,),
        in_specs=[pl.BlockSpec((_D, _K1_COLS), lambda i: (0, i))],
        out_specs=pl.BlockSpec((_K1_COLS // 2, 128), lambda i: (i, 0)),
        out_shape=jax.ShapeDtypeStruct((_V // 2, 128), jnp.float32),
    )(tT)


@functools.lru_cache(maxsize=None)
def _make_k2():
    info = plsc.get_sparse_core_info()
    nc, ns = info.num_cores, info.num_subcores
    nw = nc * ns
    assert _A % 128 == 0 and _A // 128 == nw
    per_w = 128 * _S

    mesh = plsc.VectorSubcoreMesh(core_axis_name="c", subcore_axis_name="s")

    @functools.partial(
        pl.kernel,
        out_type=jax.ShapeDtypeStruct((_S, 8, _A // 128, 8, 128), jnp.float32),
        mesh=mesh,
        scratch_types=[
            pltpu.VMEM((per_w,), jnp.int32),
            pltpu.VMEM((2, 128), jnp.int32),
            pltpu.VMEM((2, 128, _D), jnp.float32),
            pltpu.VMEM((2, 8, 8, 128), jnp.float32),
            pltpu.SemaphoreType.DMA,
            pltpu.SemaphoreType.DMA,
            pltpu.SemaphoreType.DMA,
            pltpu.SemaphoreType.DMA,
        ],
        compiler_params=pltpu.CompilerParams(
            use_tc_tiling_on_sc=False, needs_layout_passes=False
        ),
    )
    def k2(idx_hbm, r_hbm, out_hbm, idxv, pbuf, g_v, o_v, s0, s1, w0, w1):
        gsem = (s0, s1)
        wsem = (w0, w1)
        wid = lax.axis_index("s") * nc + lax.axis_index("c")
        pltpu.sync_copy(idx_hbm.at[pl.ds(wid * per_w, per_w)], idxv)

        iota = lax.iota(jnp.int32, 16)
        iota_s = iota * _S
        iota_d = iota * _D

        def extract(b, s):
            for g in range(8):
                v = plsc.load_gather(idxv, [iota_s + (g * 16 * _S + s)])
                pbuf[b, pl.ds(g * 16, 16)] = v

        def fire(b):
            pltpu.async_copy(r_hbm.at[pbuf.at[b]], g_v.at[b], gsem[b])

        def wait_gather(b):
            pltpu.make_async_copy(
                r_hbm.at[pbuf.at[b]], g_v.at[b], gsem[b]
            ).wait()

        def shuffle(b):
            for j in range(_D):
                for g in range(8):
                    vec = plsc.load_gather(g_v.at[b], [iota + g * 16,
                                                       jnp.full((16,), j, jnp.int32)])
                    o_v[b, j // 8, j % 8, pl.ds(g * 16, 16)] = vec

        def write(b, s):
            pltpu.async_copy(o_v.at[b], out_hbm.at[s, :, wid], wsem[b])

        def wait_write(b, s):
            pltpu.make_async_copy(
                o_v.at[b], out_hbm.at[s, :, wid], wsem[b]
            ).wait()

        extract(0, 0)
        fire(0)
        extract(1, 1)
        fire(1)

        def pair_body(i, carry):
            for b in range(2):
                s = 2 * i + b
                wait_gather(b)

                @pl.when(s >= 2)
                def _reuse():
                    wait_write(b, s - 2)

                shuffle(b)
                write(b, s)

                @pl.when(s + 2 < _S)
                def _next():
                    extract(b, s + 2)
                    fire(b)

            return carry

        lax.fori_loop(0, _S // 2, pair_body, 0)

        wait_write(0, _S - 2)
        wait_write(1, _S - 1)

    return k2


def kernel(input, table):
    tT = table.T
    r = _k1(tT)
    r2 = r.reshape(_V, _D)
    idxf = input.reshape(_A * _S).astype(jnp.int32)
    out5 = _make_k2()(idxf, r2)
    return out5.transpose(2, 4, 0, 1, 3).reshape(_A, _S, _D)


# speedup vs baseline: 1.6267x; 1.6267x over previous
import functools

import jax
import jax.numpy as jnp
from jax import lax
from jax.experimental import pallas as pl
from jax.experimental.pallas import tpu as pltpu
from jax.experimental.pallas import tpu_sc as plsc

_V = 1000000
_D = 64
_A = 4096
_S = 200
_SCALE = float(_D) ** 0.5
_K1_COLS = 4096


def _k1_body(t_ref, o_ref):
    blk = t_ref[...]
    t = jnp.transpose(blk, (1, 0)) * _SCALE
    t3 = jnp.reshape(t, (_K1_COLS // 2, 2, _D))
    o_ref[...] = jnp.concatenate([t3[:, 0, :], t3[:, 1, :]], axis=1)


def _k1(tT):
    n_blk = (_V + _K1_COLS - 1) // _K1_COLS
    return pl.pallas_call(
        _k1_body,
        grid=(n_blk,),
        in_specs=[pl.BlockSpec((_D, _K1_COLS), lambda i: (0, i))],
        out_specs=pl.BlockSpec((_K1_COLS // 2, 128), lambda i: (i, 0)),
        out_shape=jax.ShapeDtypeStruct((_V // 2, 128), jnp.float32),
    )(tT)


@functools.lru_cache(maxsize=None)
def _make_k2():
    info = plsc.get_sparse_core_info()
    nc, ns = info.num_cores, info.num_subcores
    nw = nc * ns
    assert _A % 128 == 0 and _A // 128 == nw
    per_w = 128 * _S

    mesh = plsc.VectorSubcoreMesh(core_axis_name="c", subcore_axis_name="s")

    @functools.partial(
        pl.kernel,
        out_type=jax.ShapeDtypeStruct((_S, 8, _A // 128, 8, 128), jnp.float32),
        mesh=mesh,
        scratch_types=[
            pltpu.VMEM((per_w,), jnp.int32),
            pltpu.VMEM((2, 128), jnp.int32),
            pltpu.VMEM((2, 128, _D), jnp.float32),
            pltpu.VMEM((2, 8, 8, 128), jnp.float32),
            pltpu.SemaphoreType.DMA,
            pltpu.SemaphoreType.DMA,
            pltpu.SemaphoreType.DMA,
            pltpu.SemaphoreType.DMA,
        ],
        compiler_params=pltpu.CompilerParams(
            use_tc_tiling_on_sc=False, needs_layout_passes=False
        ),
    )
    def k2(idx_hbm, r_hbm, out_hbm, idxv, pbuf, g_v, o_v, s0, s1, w0, w1):
        gsem = (s0, s1)
        wsem = (w0, w1)
        wid = lax.axis_index("s") * nc + lax.axis_index("c")
        pltpu.sync_copy(idx_hbm.at[pl.ds(wid * per_w, per_w)], idxv)

        iota = lax.iota(jnp.int32, 16)
        iota_s = iota * _S
        iota_d = iota * _D

        def extract(b, s):
            for g in range(8):
                v = plsc.load_gather(idxv, [iota_s + (g * 16 * _S + s)])
                pbuf[b, pl.ds(g * 16, 16)] = v

        def fire(b):
            pltpu.async_copy(r_hbm.at[pbuf.at[b]], g_v.at[b], gsem[b])

        def wait_gather(b):
            pltpu.make_async_copy(
                r_hbm.at[pbuf.at[b]], g_v.at[b], gsem[b]
            ).wait()

        def shuffle(b):
            @plsc.parallel_loop(0, _D, 1, unroll=8)
            def _jloop(j):
                tj = j // 8
                sj = lax.rem(j, 8)
                col = jnp.full((16,), j, jnp.int32)
                for g in range(8):
                    vec = plsc.load_gather(g_v.at[b], [iota + g * 16, col])
                    o_v[b, tj, sj, pl.ds(g * 16, 16)] = vec

        def write(b, s):
            pltpu.async_copy(o_v.at[b], out_hbm.at[s, :, wid], wsem[b])

        def wait_write(b, s):
            pltpu.make_async_copy(
                o_v.at[b], out_hbm.at[s, :, wid], wsem[b]
            ).wait()

        extract(0, 0)
        fire(0)
        extract(1, 1)
        fire(1)

        def pair_body(i, carry):
            for b in range(2):
                s = 2 * i + b
                wait_gather(b)

                @pl.when(s >= 2)
                def _reuse():
                    wait_write(b, s - 2)

                shuffle(b)
                write(b, s)

                @pl.when(s + 2 < _S)
                def _next():
                    extract(b, s + 2)
                    fire(b)

            return carry

        lax.fori_loop(0, _S // 2, pair_body, 0)

        wait_write(0, _S - 2)
        wait_write(1, _S - 1)

    return k2


def kernel(input, table):
    tT = table.T
    r = _k1(tT)
    r2 = r.reshape(_V, _D)
    idxf = input.reshape(_A * _S).astype(jnp.int32)
    out5 = _make_k2()(idxf, r2)
    return out5.transpose(2, 4, 0, 1, 3).reshape(_A, _S, _D)


# speedup vs baseline: 3.5182x; 2.1627x over previous
import functools

import jax
import jax.numpy as jnp
from jax import lax
from jax.experimental import pallas as pl
from jax.experimental.pallas import tpu as pltpu
from jax.experimental.pallas import tpu_sc as plsc

_V = 1000000
_D = 64
_A = 4096
_S = 200
_SCALE = float(_D) ** 0.5
_K1_COLS = 4096
_K1_HALF = _K1_COLS // 2
_K1_BLKS = (_V + _K1_COLS - 1) // _K1_COLS
_R_ROWS = _K1_BLKS * _K1_HALF


def _k1_body(t_ref, o_ref):
    blk = t_ref[...]
    o_ref[:, 0:_D] = jnp.transpose(blk[:, :_K1_HALF], (1, 0)) * _SCALE
    o_ref[:, _D:128] = jnp.transpose(blk[:, _K1_HALF:], (1, 0)) * _SCALE


def _k1(tT):
    return pl.pallas_call(
        _k1_body,
        grid=(_K1_BLKS,),
        in_specs=[pl.BlockSpec((_D, _K1_COLS), lambda i: (0, i))],
        out_specs=pl.BlockSpec((_K1_HALF, 128), lambda i: (i, 0)),
        out_shape=jax.ShapeDtypeStruct((_R_ROWS, 128), jnp.float32),
    )(tT)


@functools.lru_cache(maxsize=None)
def _make_k2():
    info = plsc.get_sparse_core_info()
    nc, ns = info.num_cores, info.num_subcores
    nw = nc * ns
    assert _A % 128 == 0 and _A // 128 == nw
    per_w = 128 * _S

    mesh = plsc.VectorSubcoreMesh(core_axis_name="c", subcore_axis_name="s")

    @functools.partial(
        pl.kernel,
        out_type=jax.ShapeDtypeStruct((_S, 8, _A // 128, 8, 128), jnp.float32),
        mesh=mesh,
        scratch_types=[
            pltpu.VMEM((per_w,), jnp.int32),
            pltpu.VMEM((2, 128), jnp.int32),
            pltpu.VMEM((2, 128, _D), jnp.float32),
            pltpu.VMEM((2, 8, 8, 128), jnp.float32),
            pltpu.SemaphoreType.DMA,
            pltpu.SemaphoreType.DMA,
            pltpu.SemaphoreType.DMA,
            pltpu.SemaphoreType.DMA,
        ],
        compiler_params=pltpu.CompilerParams(
            use_tc_tiling_on_sc=False, needs_layout_passes=False
        ),
    )
    def k2(idx_hbm, r_hbm, out_hbm, idxv, pbuf, g_v, o_v, s0, s1, w0, w1):
        gsem = (s0, s1)
        wsem = (w0, w1)
        wid = lax.axis_index("s") * nc + lax.axis_index("c")
        pltpu.sync_copy(idx_hbm.at[pl.ds(wid * per_w, per_w)], idxv)

        iota = lax.iota(jnp.int32, 16)
        iota_s = iota * _S

        def extract(b, s):
            for g in range(8):
                v = plsc.load_gather(idxv, [iota_s + (g * 16 * _S + s)])
                phys = (
                    (v & ~(_K1_COLS - 1))
                    | ((v & (_K1_HALF - 1)) << 1)
                    | ((v >> 11) & 1)
                )
                pbuf[b, pl.ds(g * 16, 16)] = phys

        def fire(b):
            pltpu.async_copy(r_hbm.at[pbuf.at[b]], g_v.at[b], gsem[b])

        def wait_gather(b):
            pltpu.make_async_copy(
                r_hbm.at[pbuf.at[b]], g_v.at[b], gsem[b]
            ).wait()

        def shuffle(b):
            @plsc.parallel_loop(0, _D, 1, unroll=8)
            def _jloop(j):
                t = (j + iota) & (_D - 1)
                tj = t >> 3
                sj = t & 7
                for g in range(8):
                    vec = plsc.load_gather(g_v.at[b], [iota + g * 16, t])
                    plsc.store_scatter(o_v.at[b], [tj, sj, iota + g * 16], vec)

        def write(b, s):
            pltpu.async_copy(o_v.at[b], out_hbm.at[s, :, wid], wsem[b])

        def wait_write(b, s):
            pltpu.make_async_copy(
                o_v.at[b], out_hbm.at[s, :, wid], wsem[b]
            ).wait()

        extract(0, 0)
        fire(0)
        extract(1, 1)
        fire(1)

        def pair_body(i, carry):
            for b in range(2):
                s = 2 * i + b
                wait_gather(b)

                @pl.when(s >= 2)
                def _reuse():
                    wait_write(b, s - 2)

                shuffle(b)
                write(b, s)

                @pl.when(s + 2 < _S)
                def _next():
                    extract(b, s + 2)
                    fire(b)

            return carry

        lax.fori_loop(0, _S // 2, pair_body, 0)

        wait_write(0, _S - 2)
        wait_write(1, _S - 1)

    return k2


def kernel(input, table):
    tT = table.T
    r = _k1(tT)
    r2 = r.reshape(2 * _R_ROWS, _D)
    idxf = input.reshape(_A * _S).astype(jnp.int32)
    out5 = _make_k2()(idxf, r2)
    return out5.transpose(2, 4, 0, 1, 3).reshape(_A, _S, _D)


# speedup vs baseline: 4.0082x; 1.1393x over previous
import functools

import jax
import jax.numpy as jnp
from jax import lax
from jax.experimental import pallas as pl
from jax.experimental.pallas import tpu as pltpu
from jax.experimental.pallas import tpu_sc as plsc

_V = 1000000
_D = 64
_A = 4096
_S = 200
_SCALE = float(_D) ** 0.5
_K1_COLS = 8192
_K1_HALF = _K1_COLS // 2
_K1_SHIFT = _K1_HALF.bit_length() - 1
_K1_BLKS = (_V + _K1_COLS - 1) // _K1_COLS
_R_ROWS = _K1_BLKS * _K1_HALF


def _k1_body(t_ref, o_ref):
    blk = t_ref[...]
    o_ref[:, 0:_D] = jnp.transpose(blk[:, :_K1_HALF], (1, 0)) * _SCALE
    o_ref[:, _D:128] = jnp.transpose(blk[:, _K1_HALF:], (1, 0)) * _SCALE


def _k1(tT):
    return pl.pallas_call(
        _k1_body,
        grid=(_K1_BLKS,),
        in_specs=[pl.BlockSpec((_D, _K1_COLS), lambda i: (0, i))],
        out_specs=pl.BlockSpec((_K1_HALF, 128), lambda i: (i, 0)),
        out_shape=jax.ShapeDtypeStruct((_R_ROWS, 128), jnp.float32),
    )(tT)


@functools.lru_cache(maxsize=None)
def _make_k2():
    info = plsc.get_sparse_core_info()
    nc, ns = info.num_cores, info.num_subcores
    nw = nc * ns
    assert _A % 128 == 0 and _A // 128 == nw
    per_w = 128 * _S

    mesh = plsc.VectorSubcoreMesh(core_axis_name="c", subcore_axis_name="s")

    @functools.partial(
        pl.kernel,
        out_type=jax.ShapeDtypeStruct((_S, 8, _A // 128, 8, 128), jnp.float32),
        mesh=mesh,
        scratch_types=[
            pltpu.VMEM((per_w,), jnp.int32),
            pltpu.VMEM((2, 128), jnp.int32),
            pltpu.VMEM((2, 128, _D), jnp.float32),
            pltpu.VMEM((2, 8, 8, 128), jnp.float32),
            pltpu.SemaphoreType.DMA,
            pltpu.SemaphoreType.DMA,
            pltpu.SemaphoreType.DMA,
            pltpu.SemaphoreType.DMA,
        ],
        compiler_params=pltpu.CompilerParams(
            use_tc_tiling_on_sc=False, needs_layout_passes=False
        ),
    )
    def k2(idx_hbm, r_hbm, out_hbm, idxv, pbuf, g_v, o_v, s0, s1, w0, w1):
        gsem = (s0, s1)
        wsem = (w0, w1)
        wid = lax.axis_index("s") * nc + lax.axis_index("c")
        pltpu.sync_copy(idx_hbm.at[pl.ds(wid * per_w, per_w)], idxv)

        iota = lax.iota(jnp.int32, 16)
        iota_s = iota * _S

        def extract(b, s):
            for g in range(8):
                v = plsc.load_gather(idxv, [iota_s + (g * 16 * _S + s)])
                phys = (
                    (v & ~(_K1_COLS - 1))
                    | ((v & (_K1_HALF - 1)) << 1)
                    | ((v >> _K1_SHIFT) & 1)
                )
                pbuf[b, pl.ds(g * 16, 16)] = phys

        def fire(b):
            pltpu.async_copy(r_hbm.at[pbuf.at[b]], g_v.at[b], gsem[b])

        def wait_gather(b):
            pltpu.make_async_copy(
                r_hbm.at[pbuf.at[b]], g_v.at[b], gsem[b]
            ).wait()

        def shuffle(b):
            @plsc.parallel_loop(0, _D, 1, unroll=8)
            def _jloop(j):
                t = (j + iota) & (_D - 1)
                tj = t >> 3
                sj = t & 7
                for g in range(8):
                    vec = plsc.load_gather(g_v.at[b], [iota + g * 16, t])
                    plsc.store_scatter(o_v.at[b], [tj, sj, iota + g * 16], vec)

        def write(b, s):
            pltpu.async_copy(o_v.at[b], out_hbm.at[s, :, wid], wsem[b])

        def wait_write(b, s):
            pltpu.make_async_copy(
                o_v.at[b], out_hbm.at[s, :, wid], wsem[b]
            ).wait()

        extract(0, 0)
        fire(0)
        extract(1, 1)
        fire(1)

        def pair_body(i, carry):
            for b in range(2):
                s = 2 * i + b
                wait_gather(b)

                @pl.when(s >= 2)
                def _reuse():
                    wait_write(b, s - 2)

                shuffle(b)
                write(b, s)

                @pl.when(s + 2 < _S)
                def _next():
                    extract(b, s + 2)
                    fire(b)

            return carry

        lax.fori_loop(0, _S // 2, pair_body, 0)

        wait_write(0, _S - 2)
        wait_write(1, _S - 1)

    return k2


def kernel(input, table):
    tT = table.T
    r = _k1(tT)
    r2 = r.reshape(2 * _R_ROWS, _D)
    idxf = input.reshape(_A * _S).astype(jnp.int32)
    out5 = _make_k2()(idxf, r2)
    return out5.transpose(2, 4, 0, 1, 3).reshape(_A, _S, _D)
